# Initial kernel scaffold; baseline (speedup 1.0000x reference)
import jax
import jax.numpy as jnp
from jax.experimental import pallas as pl


def kernel(x, W, spectrum, row_idx, col_idx):
    raise NotImplementedError("write your pallas kernel here")


# speedup vs baseline: 3.5545x; 3.5545x over previous
import functools

import jax
import jax.numpy as jnp
from jax import lax
from jax.experimental import pallas as pl
from jax.experimental.pallas import tpu as pltpu
from jax.experimental.pallas import tpu_sc as plsc

VOCAB = 100000
DIM = 128
SCALING = 1.0
NPAD = 1008
NC, NS, L = 2, 16, 16
NW = NC * NS
CHUNK = 128


def _wid():
    return lax.axis_index("s") * NC + lax.axis_index("c")


def _mesh():
    return plsc.VectorSubcoreMesh(core_axis_name="c", subcore_axis_name="s")


_CP = pltpu.CompilerParams(needs_layout_passes=False, use_tc_tiling_on_sc=False)


@functools.partial(
    pl.kernel,
    out_type=jax.ShapeDtypeStruct((VOCAB,), jnp.int32),
    mesh=_mesh(),
    compiler_params=_CP,
    scratch_types=[
        pltpu.VMEM((VOCAB,), jnp.int32),
        pltpu.VMEM((NPAD,), jnp.int32),
        pltpu.VMEM((NPAD,), jnp.int32),
    ],
)
def _build_rid(srow_hbm, pval_hbm, rid_hbm, rid_v, srow_v, pval_v):
    @pl.when(_wid() == 0)
    def _():
        def zero_body(i, c):
            rid_v[pl.ds(i * L, L)] = jnp.zeros((L,), jnp.int32)
            return c

        lax.fori_loop(0, VOCAB // L, zero_body, 0)
        pltpu.sync_copy(srow_hbm, srow_v)
        pltpu.sync_copy(pval_hbm, pval_v)

        def scat_body(i, c):
            off = i * L
            rows = srow_v[pl.ds(off, L)]
            vals = pval_v[pl.ds(off, L)]
            msk = (lax.iota(jnp.int32, L) + off) < srow_v.shape[0]
            plsc.store_scatter(rid_v, [rows], vals, mask=msk)
            return c

        lax.fori_loop(0, NPAD // L, scat_body, 0)
        pltpu.sync_copy(rid_v, rid_hbm)


def _make_gather_add(n_tok):
    b_per_w = n_tok // NW
    n_chunks = b_per_w // CHUNK

    @functools.partial(
        pl.kernel,
        out_type=jax.ShapeDtypeStruct((n_tok, DIM), jnp.float32),
        mesh=_mesh(),
        compiler_params=_CP,
        scratch_types=[
            pltpu.VMEM((CHUNK,), jnp.int32),
            pltpu.VMEM((CHUNK,), jnp.int32),
            pltpu.VMEM((CHUNK, DIM), jnp.float32),
            pltpu.VMEM((NPAD,), jnp.int32),
            pltpu.VMEM((NPAD,), jnp.float32),
            pltpu.SemaphoreType.DMA,
            pltpu.SemaphoreType.DMA,
        ],
    )
    def gather_add(xf_hbm, rid_hbm, w_hbm, scol_hbm, sval_hbm, out_hbm,
                   idx_v, ridv_v, rows_v, scol_v, sval_v, sem_w, sem_r):
        wid = _wid()
        pltpu.sync_copy(scol_hbm, scol_v)
        pltpu.sync_copy(sval_hbm, sval_v)

        def chunk_body(g, c):
            base = wid * b_per_w + g * CHUNK
            pltpu.sync_copy(xf_hbm.at[pl.ds(base, CHUNK)], idx_v)
            cp_w = pltpu.async_copy(w_hbm.at[idx_v], rows_v, sem_w)
            cp_r = pltpu.async_copy(rid_hbm.at[idx_v], ridv_v, sem_r)
            cp_r.wait()
            cp_w.wait()
            for v in range(CHUNK // L):
                ridv = ridv_v[pl.ds(v * L, L)]
                cnt = lax.bitwise_and(ridv, 2047)
                start = lax.shift_right_logical(ridv, 11)
                m = jnp.max(cnt)
                lanes = lax.iota(jnp.int32, L) + (v * L)

                def delta_body(k, cc, start=start, cnt=cnt, lanes=lanes):
                    msk = k < cnt
                    j = jnp.minimum(start + k, NPAD - 1)
                    col = plsc.load_gather(scol_v, [j], mask=msk)
                    val = plsc.load_gather(sval_v, [j], mask=msk)
                    plsc.addupdate_scatter(rows_v, [lanes, col], val, mask=msk)
                    return cc

                lax.fori_loop(0, m, delta_body, 0)
            pltpu.sync_copy(rows_v, out_hbm.at[pl.ds(base, CHUNK)])
            return c

        lax.fori_loop(0, n_chunks, chunk_body, 0)

    return gather_add


def kernel(x, W, spectrum, row_idx, col_idx):
    bsz, hist = x.shape
    n_tok = bsz * hist
    xf = x.reshape(n_tok).astype(jnp.int32)
    n_freq = row_idx.shape[0]

    order = jnp.argsort(row_idx)
    srow = jnp.take(row_idx, order).astype(jnp.int32)
    scol = jnp.take(col_idx, order).astype(jnp.int32)
    sval = jnp.take(spectrum.astype(jnp.float32) * SCALING, order)
    first = jnp.searchsorted(srow, srow, side="left").astype(jnp.int32)
    last = jnp.searchsorted(srow, srow, side="right").astype(jnp.int32)
    packed = first * 2048 + (last - first)

    pad = NPAD - n_freq
    srow_p = jnp.pad(srow, (0, pad))
    packed_p = jnp.pad(packed, (0, pad))
    scol_p = jnp.pad(scol, (0, pad))
    sval_p = jnp.pad(sval, (0, pad))

    rid = _build_rid(srow_p, packed_p)
    out = _make_gather_add(n_tok)(xf, rid, W, scol_p, sval_p)
    return out.reshape(bsz, hist, DIM)


# speedup vs baseline: 8.4068x; 2.3651x over previous
import functools

import jax
import jax.numpy as jnp
from jax import lax
from jax.experimental import pallas as pl
from jax.experimental.pallas import tpu as pltpu
from jax.experimental.pallas import tpu_sc as plsc

VOCAB = 100000
DIM = 128
SCALING = 1.0
NPAD = 1008
NC, NS, L = 2, 16, 16
NW = NC * NS
CHUNK = 128


def _wid():
    return lax.axis_index("s") * NC + lax.axis_index("c")


def _mesh():
    return plsc.VectorSubcoreMesh(core_axis_name="c", subcore_axis_name="s")


_CP = pltpu.CompilerParams(needs_layout_passes=False, use_tc_tiling_on_sc=False)


@functools.partial(
    pl.kernel,
    out_type=jax.ShapeDtypeStruct((VOCAB,), jnp.int32),
    mesh=_mesh(),
    compiler_params=_CP,
    scratch_types=[
        pltpu.VMEM((VOCAB,), jnp.int32),
        pltpu.VMEM((NPAD,), jnp.int32),
        pltpu.VMEM((NPAD,), jnp.int32),
    ],
)
def _build_rid(srow_hbm, pval_hbm, rid_hbm, rid_v, srow_v, pval_v):
    @pl.when(_wid() == 0)
    def _():
        def zero_body(i, c):
            rid_v[pl.ds(i * L, L)] = jnp.zeros((L,), jnp.int32)
            return c

        lax.fori_loop(0, VOCAB // L, zero_body, 0)
        pltpu.sync_copy(srow_hbm, srow_v)
        pltpu.sync_copy(pval_hbm, pval_v)

        def scat_body(i, c):
            off = i * L
            rows = srow_v[pl.ds(off, L)]
            vals = pval_v[pl.ds(off, L)]
            msk = (lax.iota(jnp.int32, L) + off) < srow_v.shape[0]
            plsc.store_scatter(rid_v, [rows], vals, mask=msk)
            return c

        lax.fori_loop(0, NPAD // L, scat_body, 0)
        pltpu.sync_copy(rid_v, rid_hbm)


def _make_gather_add(n_tok):
    b_per_w = n_tok // NW
    n_chunks = b_per_w // CHUNK

    @functools.partial(
        pl.kernel,
        out_type=jax.ShapeDtypeStruct((n_tok, DIM), jnp.float32),
        mesh=_mesh(),
        compiler_params=_CP,
        scratch_types=[
            pltpu.VMEM((CHUNK,), jnp.int32),
            pltpu.VMEM((CHUNK,), jnp.int32),
            pltpu.VMEM((CHUNK, DIM), jnp.float32),
            pltpu.VMEM((NPAD,), jnp.int32),
            pltpu.VMEM((NPAD,), jnp.float32),
            pltpu.SemaphoreType.DMA,
            pltpu.SemaphoreType.DMA,
        ],
    )
    def gather_add(xf_hbm, rid_hbm, w_hbm, scol_hbm, sval_hbm, out_hbm,
                   idx_v, ridv_v, rows_v, scol_v, sval_v, sem_w, sem_r):
        wid = _wid()
        pltpu.sync_copy(scol_hbm, scol_v)
        pltpu.sync_copy(sval_hbm, sval_v)

        def chunk_body(g, c):
            base = wid * b_per_w + g * CHUNK
            pltpu.sync_copy(xf_hbm.at[pl.ds(base, CHUNK)], idx_v)
            cp_w = pltpu.async_copy(w_hbm.at[idx_v], rows_v, sem_w)
            cp_r = pltpu.async_copy(rid_hbm.at[idx_v], ridv_v, sem_r)
            cp_r.wait()
            cp_w.wait()
            for v in range(CHUNK // L):
                ridv = ridv_v[pl.ds(v * L, L)]
                cnt = lax.bitwise_and(ridv, 2047)
                start = lax.shift_right_logical(ridv, 11)
                m = jnp.max(cnt)
                lanes = lax.iota(jnp.int32, L) + (v * L)

                def delta_body(k, cc, start=start, cnt=cnt, lanes=lanes):
                    msk = k < cnt
                    j = jnp.minimum(start + k, NPAD - 1)
                    col = plsc.load_gather(scol_v, [j], mask=msk)
                    val = plsc.load_gather(sval_v, [j], mask=msk)
                    plsc.addupdate_scatter(rows_v, [lanes, col], val, mask=msk)
                    return cc

                lax.fori_loop(0, m, delta_body, 0)
            pltpu.sync_copy(rows_v, out_hbm.at[pl.ds(base, CHUNK)])
            return c

        lax.fori_loop(0, n_chunks, chunk_body, 0)

    return gather_add


def kernel(x, W, spectrum, row_idx, col_idx):
    bsz, hist = x.shape
    n_tok = bsz * hist
    xf = jnp.transpose(x).reshape(n_tok).astype(jnp.int32)
    n_freq = row_idx.shape[0]

    order = jnp.argsort(row_idx)
    srow = jnp.take(row_idx, order).astype(jnp.int32)
    scol = jnp.take(col_idx, order).astype(jnp.int32)
    sval = jnp.take(spectrum.astype(jnp.float32) * SCALING, order)
    iota = jnp.arange(n_freq, dtype=jnp.int32)
    is_start = jnp.concatenate([jnp.ones((1,), bool), srow[1:] != srow[:-1]])
    first = lax.cummax(jnp.where(is_start, iota, 0))
    is_end = jnp.concatenate([srow[:-1] != srow[1:], jnp.ones((1,), bool)])
    last = jnp.flip(lax.cummin(jnp.flip(jnp.where(is_end, iota, n_freq - 1))))
    packed = first * 2048 + (last - first + 1)

    pad = NPAD - n_freq
    srow_p = jnp.pad(srow, (0, pad))
    packed_p = jnp.pad(packed, (0, pad))
    scol_p = jnp.pad(scol, (0, pad))
    sval_p = jnp.pad(sval, (0, pad))

    rid = _build_rid(srow_p, packed_p)
    out = _make_gather_add(n_tok)(xf, rid, W, scol_p, sval_p)
    return jnp.transpose(out.reshape(hist, bsz, DIM), (1, 0, 2))


# speedup vs baseline: 11.8588x; 1.4106x over previous
import functools

import jax
import jax.numpy as jnp
from jax import lax
from jax.experimental import pallas as pl
from jax.experimental.pallas import tpu as pltpu
from jax.experimental.pallas import tpu_sc as plsc

VOCAB = 100000
DIM = 128
SCALING = 1.0
NPAD = 1008
NC, NS, L = 2, 16, 16
NW = NC * NS
CHUNK = 128


def _wid():
    return lax.axis_index("s") * NC + lax.axis_index("c")


def _mesh():
    return plsc.VectorSubcoreMesh(core_axis_name="c", subcore_axis_name="s")


_CP = pltpu.CompilerParams(needs_layout_passes=False, use_tc_tiling_on_sc=False)


@functools.partial(
    pl.kernel,
    out_type=jax.ShapeDtypeStruct((VOCAB,), jnp.int32),
    mesh=_mesh(),
    compiler_params=_CP,
    scratch_types=[
        pltpu.VMEM((VOCAB,), jnp.int32),
        pltpu.VMEM((NPAD,), jnp.int32),
        pltpu.VMEM((NPAD,), jnp.int32),
    ],
)
def _build_rid(srow_hbm, pval_hbm, rid_hbm, rid_v, srow_v, pval_v):
    @pl.when(_wid() == 0)
    def _():
        def zero_body(i, c):
            rid_v[pl.ds(i * L, L)] = jnp.zeros((L,), jnp.int32)
            return c

        lax.fori_loop(0, VOCAB // L, zero_body, 0)
        pltpu.sync_copy(srow_hbm, srow_v)
        pltpu.sync_copy(pval_hbm, pval_v)

        def scat_body(i, c):
            off = i * L
            rows = srow_v[pl.ds(off, L)]
            vals = pval_v[pl.ds(off, L)]
            msk = (lax.iota(jnp.int32, L) + off) < srow_v.shape[0]
            plsc.store_scatter(rid_v, [rows], vals, mask=msk)
            return c

        lax.fori_loop(0, NPAD // L, scat_body, 0)
        pltpu.sync_copy(rid_v, rid_hbm)


def _make_gather_add(n_tok):
    b_per_w = n_tok // NW
    n_chunks = b_per_w // CHUNK

    @functools.partial(
        pl.kernel,
        out_type=jax.ShapeDtypeStruct((n_tok, DIM), jnp.float32),
        mesh=_mesh(),
        compiler_params=_CP,
        scratch_types=[
            pltpu.VMEM((n_chunks, CHUNK), jnp.int32),
            pltpu.VMEM((CHUNK,), jnp.int32),
            pltpu.VMEM((CHUNK,), jnp.int32),
            pltpu.VMEM((CHUNK, DIM), jnp.float32),
            pltpu.VMEM((CHUNK, DIM), jnp.float32),
            pltpu.VMEM((NPAD,), jnp.int32),
            pltpu.VMEM((NPAD,), jnp.float32),
            pltpu.SemaphoreType.DMA,
            pltpu.SemaphoreType.DMA,
            pltpu.SemaphoreType.DMA,
            pltpu.SemaphoreType.DMA,
            pltpu.SemaphoreType.DMA,
            pltpu.SemaphoreType.DMA,
        ],
    )
    def gather_add(xf2_hbm, rid_hbm, w_hbm, scol_hbm, sval_hbm, out_hbm,
                   idx2_v, ridv0, ridv1, rows0, rows1, scol_v, sval_v,
                   sw0, sw1, sr0, sr1, so0, so1):
        wid = _wid()
        rows = (rows0, rows1)
        ridv = (ridv0, ridv1)
        sw = (sw0, sw1)
        sr = (sr0, sr1)
        so = (so0, so1)
        pltpu.sync_copy(scol_hbm, scol_v)
        pltpu.sync_copy(sval_hbm, sval_v)
        pltpu.sync_copy(xf2_hbm.at[pl.ds(wid * n_chunks, n_chunks)], idx2_v)
        out_base = wid * b_per_w

        def start_gather(t, b):
            @pl.when(t < n_chunks)
            def _():
                @pl.when(t >= 2)
                def _():
                    pltpu.make_async_copy(
                        rows[b], out_hbm.at[pl.ds(out_base, CHUNK)], so[b]
                    ).wait()
                pltpu.async_copy(w_hbm.at[idx2_v.at[t]], rows[b], sw[b])
                pltpu.async_copy(rid_hbm.at[idx2_v.at[t]], ridv[b], sr[b])

        def finish_chunk(t, b):
            @pl.when(jnp.logical_and(t >= 0, t < n_chunks))
            def _():
                pltpu.make_async_copy(
                    w_hbm.at[idx2_v.at[0]], rows[b], sw[b]
                ).wait()
                pltpu.make_async_copy(
                    rid_hbm.at[idx2_v.at[0]], ridv[b], sr[b]
                ).wait()
                for v in range(CHUNK // L):
                    rv = ridv[b][pl.ds(v * L, L)]
                    cnt = lax.bitwise_and(rv, 2047)
                    start = lax.shift_right_logical(rv, 11)
                    m = jnp.max(cnt)
                    lanes = lax.iota(jnp.int32, L) + (v * L)

                    def delta_body(k, cc, start=start, cnt=cnt, lanes=lanes, b=b):
                        msk = k < cnt
                        j = jnp.minimum(start + k, NPAD - 1)
                        col = plsc.load_gather(scol_v, [j], mask=msk)
                        val = plsc.load_gather(sval_v, [j], mask=msk)
                        plsc.addupdate_scatter(rows[b], [lanes, col], val, mask=msk)
                        return cc

                    lax.fori_loop(0, m, delta_body, 0)
                pltpu.async_copy(
                    rows[b], out_hbm.at[pl.ds(out_base + t * CHUNK, CHUNK)], so[b]
                )

        def turn(i, c):
            for b in range(2):
                t = 2 * i + b
                start_gather(t, b)
                finish_chunk(t - 1, 1 - b)
            return c

        lax.fori_loop(0, (n_chunks + 2) // 2, turn, 0)
        pltpu.make_async_copy(rows0, out_hbm.at[pl.ds(out_base, CHUNK)], so0).wait()
        pltpu.make_async_copy(rows1, out_hbm.at[pl.ds(out_base, CHUNK)], so1).wait()

    return gather_add


def kernel(x, W, spectrum, row_idx, col_idx):
    bsz, hist = x.shape
    n_tok = bsz * hist
    xf = jnp.transpose(x).reshape(n_tok).astype(jnp.int32)
    n_freq = row_idx.shape[0]

    order = jnp.argsort(row_idx)
    srow = jnp.take(row_idx, order).astype(jnp.int32)
    scol = jnp.take(col_idx, order).astype(jnp.int32)
    sval = jnp.take(spectrum.astype(jnp.float32) * SCALING, order)
    iota = jnp.arange(n_freq, dtype=jnp.int32)
    is_start = jnp.concatenate([jnp.ones((1,), bool), srow[1:] != srow[:-1]])
    first = lax.cummax(jnp.where(is_start, iota, 0))
    is_end = jnp.concatenate([srow[:-1] != srow[1:], jnp.ones((1,), bool)])
    last = jnp.flip(lax.cummin(jnp.flip(jnp.where(is_end, iota, n_freq - 1))))
    packed = first * 2048 + (last - first + 1)

    pad = NPAD - n_freq
    srow_p = jnp.pad(srow, (0, pad))
    packed_p = jnp.pad(packed, (0, pad))
    scol_p = jnp.pad(scol, (0, pad))
    sval_p = jnp.pad(sval, (0, pad))

    rid = _build_rid(srow_p, packed_p)
    xf2 = xf.reshape(n_tok // CHUNK, CHUNK)
    out = _make_gather_add(n_tok)(xf2, rid, W, scol_p, sval_p)
    return jnp.transpose(out.reshape(hist, bsz, DIM), (1, 0, 2))


# speedup vs baseline: 14.3016x; 1.2060x over previous
import functools

import jax
import jax.numpy as jnp
from jax import lax
from jax.experimental import pallas as pl
from jax.experimental.pallas import tpu as pltpu
from jax.experimental.pallas import tpu_sc as plsc

VOCAB = 100000
VOCAB_PAD = 102400
DIM = 128
SCALING = 1.0
NPAD = 1008
NC, NS, L = 2, 16, 16
NW = NC * NS
CHUNK = 128


def _wid():
    return lax.axis_index("s") * NC + lax.axis_index("c")


def _mesh():
    return plsc.VectorSubcoreMesh(core_axis_name="c", subcore_axis_name="s")


_CP = pltpu.CompilerParams(needs_layout_passes=False, use_tc_tiling_on_sc=False)


_RID_SLICE = VOCAB_PAD // NW


@functools.partial(
    pl.kernel,
    out_type=jax.ShapeDtypeStruct((VOCAB_PAD,), jnp.int32),
    mesh=_mesh(),
    compiler_params=_CP,
    scratch_types=[
        pltpu.VMEM((_RID_SLICE,), jnp.int32),
        pltpu.VMEM((NPAD,), jnp.int32),
        pltpu.VMEM((NPAD,), jnp.int32),
    ],
)
def _build_rid(srow_hbm, pval_hbm, rid_hbm, rid_v, srow_v, pval_v):
    lo = _wid() * _RID_SLICE

    def zero_body(i, c):
        rid_v[pl.ds(i * L, L)] = jnp.zeros((L,), jnp.int32)
        return c

    lax.fori_loop(0, _RID_SLICE // L, zero_body, 0)
    pltpu.sync_copy(srow_hbm, srow_v)
    pltpu.sync_copy(pval_hbm, pval_v)

    def scat_body(i, c):
        off = i * L
        rows = srow_v[pl.ds(off, L)] - lo
        vals = pval_v[pl.ds(off, L)]
        msk = (rows >= 0) & (rows < _RID_SLICE)
        plsc.store_scatter(rid_v, [rows], vals, mask=msk)
        return c

    lax.fori_loop(0, NPAD // L, scat_body, 0)
    pltpu.sync_copy(rid_v, rid_hbm.at[pl.ds(lo, _RID_SLICE)])


def _make_gather_add(n_tok):
    b_per_w = n_tok // NW
    n_chunks = b_per_w // CHUNK

    @functools.partial(
        pl.kernel,
        out_type=jax.ShapeDtypeStruct((n_tok, DIM), jnp.float32),
        mesh=_mesh(),
        compiler_params=_CP,
        scratch_types=[
            pltpu.VMEM((n_chunks, CHUNK), jnp.int32),
            pltpu.VMEM((CHUNK,), jnp.int32),
            pltpu.VMEM((CHUNK,), jnp.int32),
            pltpu.VMEM((CHUNK, DIM), jnp.float32),
            pltpu.VMEM((CHUNK, DIM), jnp.float32),
            pltpu.VMEM((NPAD,), jnp.int32),
            pltpu.VMEM((NPAD,), jnp.float32),
            pltpu.SemaphoreType.DMA,
            pltpu.SemaphoreType.DMA,
            pltpu.SemaphoreType.DMA,
            pltpu.SemaphoreType.DMA,
            pltpu.SemaphoreType.DMA,
            pltpu.SemaphoreType.DMA,
        ],
    )
    def gather_add(xf2_hbm, rid_hbm, w_hbm, scol_hbm, sval_hbm, out_hbm,
                   idx2_v, ridv0, ridv1, rows0, rows1, scol_v, sval_v,
                   sw0, sw1, sr0, sr1, so0, so1):
        wid = _wid()
        rows = (rows0, rows1)
        ridv = (ridv0, ridv1)
        sw = (sw0, sw1)
        sr = (sr0, sr1)
        so = (so0, so1)
        pltpu.sync_copy(scol_hbm, scol_v)
        pltpu.sync_copy(sval_hbm, sval_v)
        pltpu.sync_copy(xf2_hbm.at[pl.ds(wid * n_chunks, n_chunks)], idx2_v)
        out_base = wid * b_per_w

        def start_gather(t, b):
            @pl.when(t < n_chunks)
            def _():
                @pl.when(t >= 2)
                def _():
                    pltpu.make_async_copy(
                        rows[b], out_hbm.at[pl.ds(out_base, CHUNK)], so[b]
                    ).wait()
                pltpu.async_copy(w_hbm.at[idx2_v.at[t]], rows[b], sw[b])
                pltpu.async_copy(rid_hbm.at[idx2_v.at[t]], ridv[b], sr[b])

        def finish_chunk(t, b):
            @pl.when(jnp.logical_and(t >= 0, t < n_chunks))
            def _():
                pltpu.make_async_copy(
                    w_hbm.at[idx2_v.at[0]], rows[b], sw[b]
                ).wait()
                pltpu.make_async_copy(
                    rid_hbm.at[idx2_v.at[0]], ridv[b], sr[b]
                ).wait()
                for v in range(CHUNK // L):
                    rv = ridv[b][pl.ds(v * L, L)]
                    cnt = lax.bitwise_and(rv, 2047)
                    start = lax.shift_right_logical(rv, 11)
                    m = jnp.max(cnt)
                    lanes = lax.iota(jnp.int32, L) + (v * L)

                    def delta_body(k, cc, start=start, cnt=cnt, lanes=lanes, b=b):
                        msk = k < cnt
                        j = jnp.minimum(start + k, NPAD - 1)
                        col = plsc.load_gather(scol_v, [j], mask=msk)
                        val = plsc.load_gather(sval_v, [j], mask=msk)
                        plsc.addupdate_scatter(rows[b], [lanes, col], val, mask=msk)
                        return cc

                    lax.fori_loop(0, m, delta_body, 0)
                pltpu.async_copy(
                    rows[b], out_hbm.at[pl.ds(out_base + t * CHUNK, CHUNK)], so[b]
                )

        def turn(i, c):
            for b in range(2):
                t = 2 * i + b
                start_gather(t, b)
                finish_chunk(t - 1, 1 - b)
            return c

        lax.fori_loop(0, (n_chunks + 2) // 2, turn, 0)
        pltpu.make_async_copy(rows0, out_hbm.at[pl.ds(out_base, CHUNK)], so0).wait()
        pltpu.make_async_copy(rows1, out_hbm.at[pl.ds(out_base, CHUNK)], so1).wait()

    return gather_add


def kernel(x, W, spectrum, row_idx, col_idx):
    bsz, hist = x.shape
    n_tok = bsz * hist
    xf = jnp.transpose(x).reshape(n_tok).astype(jnp.int32)
    n_freq = row_idx.shape[0]

    order = jnp.argsort(row_idx)
    srow = jnp.take(row_idx, order).astype(jnp.int32)
    scol = jnp.take(col_idx, order).astype(jnp.int32)
    sval = jnp.take(spectrum.astype(jnp.float32) * SCALING, order)
    iota = jnp.arange(n_freq, dtype=jnp.int32)
    is_start = jnp.concatenate([jnp.ones((1,), bool), srow[1:] != srow[:-1]])
    first = lax.cummax(jnp.where(is_start, iota, 0))
    is_end = jnp.concatenate([srow[:-1] != srow[1:], jnp.ones((1,), bool)])
    last = jnp.flip(lax.cummin(jnp.flip(jnp.where(is_end, iota, n_freq - 1))))
    packed = first * 2048 + (last - first + 1)

    pad = NPAD - n_freq
    srow_p = jnp.pad(srow, (0, pad), constant_values=VOCAB_PAD)
    packed_p = jnp.pad(packed, (0, pad))
    scol_p = jnp.pad(scol, (0, pad))
    sval_p = jnp.pad(sval, (0, pad))

    rid = _build_rid(srow_p, packed_p)
    xf2 = xf.reshape(n_tok // CHUNK, CHUNK)
    out = _make_gather_add(n_tok)(xf2, rid, W, scol_p, sval_p)
    return jnp.transpose(out.reshape(hist, bsz, DIM), (1, 0, 2))


# speedup vs baseline: 16.4278x; 1.1487x over previous
import functools

import jax
import jax.numpy as jnp
from jax import lax
from jax.experimental import pallas as pl
from jax.experimental.pallas import tpu as pltpu
from jax.experimental.pallas import tpu_sc as plsc

VOCAB = 100000
VOCAB_PAD = 102400
DIM = 128
SCALING = 1.0
NPAD = 1024
NC, NS, L = 2, 16, 16
NW = NC * NS
CHUNK = 128


def _wid():
    return lax.axis_index("s") * NC + lax.axis_index("c")


def _mesh():
    return plsc.VectorSubcoreMesh(core_axis_name="c", subcore_axis_name="s")


_CP = pltpu.CompilerParams(needs_layout_passes=False, use_tc_tiling_on_sc=False)


_RID_SLICE = VOCAB_PAD // NW


@functools.partial(
    pl.kernel,
    out_type=(
        jax.ShapeDtypeStruct((VOCAB_PAD,), jnp.int32),
        jax.ShapeDtypeStruct((NPAD,), jnp.int32),
        jax.ShapeDtypeStruct((NPAD,), jnp.float32),
    ),
    mesh=_mesh(),
    compiler_params=_CP,
    scratch_types=[
        pltpu.VMEM((_RID_SLICE,), jnp.int32),
        pltpu.VMEM((NPAD,), jnp.int32),
        pltpu.VMEM((NPAD,), jnp.int32),
        pltpu.VMEM((CHUNK,), jnp.int32),
        pltpu.VMEM((CHUNK,), jnp.int32),
        pltpu.VMEM((CHUNK,), jnp.float32),
        pltpu.SemaphoreType.DMA,
        pltpu.SemaphoreType.DMA,
    ],
)
def _build_rid(srow_hbm, pval_hbm, order_hbm, col_hbm, val_hbm,
               rid_hbm, scol_hbm, sval_hbm,
               rid_v, srow_v, pval_v, ord_v, colg_v, valg_v, sem0, sem1):
    wid = _wid()
    lo = wid * _RID_SLICE

    @pl.when(wid < NPAD // CHUNK)
    def _():
        pltpu.sync_copy(order_hbm.at[pl.ds(wid * CHUNK, CHUNK)], ord_v)
        pltpu.async_copy(col_hbm.at[ord_v], colg_v, sem0)
        pltpu.async_copy(val_hbm.at[ord_v], valg_v, sem1)

    def zero_body(i, c):
        rid_v[pl.ds(i * L, L)] = jnp.zeros((L,), jnp.int32)
        return c

    lax.fori_loop(0, _RID_SLICE // L, zero_body, 0)
    pltpu.sync_copy(srow_hbm, srow_v)
    pltpu.sync_copy(pval_hbm, pval_v)

    @pl.when(wid < NPAD // CHUNK)
    def _():
        pltpu.make_async_copy(col_hbm.at[ord_v], colg_v, sem0).wait()
        pltpu.make_async_copy(val_hbm.at[ord_v], valg_v, sem1).wait()
        pltpu.sync_copy(colg_v, scol_hbm.at[pl.ds(wid * CHUNK, CHUNK)])
        pltpu.sync_copy(valg_v, sval_hbm.at[pl.ds(wid * CHUNK, CHUNK)])

    def scat_body(i, c):
        off = i * L
        rows = srow_v[pl.ds(off, L)] - lo
        vals = pval_v[pl.ds(off, L)]
        msk = (rows >= 0) & (rows < _RID_SLICE)
        plsc.store_scatter(rid_v, [rows], vals, mask=msk)
        return c

    lax.fori_loop(0, NPAD // L, scat_body, 0)
    pltpu.sync_copy(rid_v, rid_hbm.at[pl.ds(lo, _RID_SLICE)])


def _make_gather_add(n_tok):
    b_per_w = n_tok // NW
    n_chunks = b_per_w // CHUNK

    @functools.partial(
        pl.kernel,
        out_type=jax.ShapeDtypeStruct((n_tok, DIM), jnp.float32),
        mesh=_mesh(),
        compiler_params=_CP,
        scratch_types=[
            pltpu.VMEM((n_chunks, CHUNK), jnp.int32),
            pltpu.VMEM((CHUNK,), jnp.int32),
            pltpu.VMEM((CHUNK,), jnp.int32),
            pltpu.VMEM((CHUNK, DIM), jnp.float32),
            pltpu.VMEM((CHUNK, DIM), jnp.float32),
            pltpu.VMEM((NPAD,), jnp.int32),
            pltpu.VMEM((NPAD,), jnp.float32),
            pltpu.SemaphoreType.DMA,
            pltpu.SemaphoreType.DMA,
            pltpu.SemaphoreType.DMA,
            pltpu.SemaphoreType.DMA,
            pltpu.SemaphoreType.DMA,
            pltpu.SemaphoreType.DMA,
        ],
    )
    def gather_add(xf2_hbm, rid_hbm, w_hbm, scol_hbm, sval_hbm, out_hbm,
                   idx2_v, ridv0, ridv1, rows0, rows1, scol_v, sval_v,
                   sw0, sw1, sr0, sr1, so0, so1):
        wid = _wid()
        rows = (rows0, rows1)
        ridv = (ridv0, ridv1)
        sw = (sw0, sw1)
        sr = (sr0, sr1)
        so = (so0, so1)
        pltpu.sync_copy(scol_hbm, scol_v)
        pltpu.sync_copy(sval_hbm, sval_v)
        pltpu.sync_copy(xf2_hbm.at[pl.ds(wid * n_chunks, n_chunks)], idx2_v)
        out_base = wid * b_per_w

        def start_gather(t, b):
            @pl.when(t < n_chunks)
            def _():
                @pl.when(t >= 2)
                def _():
                    pltpu.make_async_copy(
                        rows[b], out_hbm.at[pl.ds(out_base, CHUNK)], so[b]
                    ).wait()
                pltpu.async_copy(w_hbm.at[idx2_v.at[t]], rows[b], sw[b])
                pltpu.async_copy(rid_hbm.at[idx2_v.at[t]], ridv[b], sr[b])

        def finish_chunk(t, b):
            @pl.when(jnp.logical_and(t >= 0, t < n_chunks))
            def _():
                pltpu.make_async_copy(
                    w_hbm.at[idx2_v.at[0]], rows[b], sw[b]
                ).wait()
                pltpu.make_async_copy(
                    rid_hbm.at[idx2_v.at[0]], ridv[b], sr[b]
                ).wait()
                for v in range(CHUNK // L):
                    rv = ridv[b][pl.ds(v * L, L)]
                    cnt = lax.bitwise_and(rv, 2047)
                    start = lax.shift_right_logical(rv, 11)
                    m = jnp.max(cnt)
                    lanes = lax.iota(jnp.int32, L) + (v * L)

                    def delta_body(k, cc, start=start, cnt=cnt, lanes=lanes, b=b):
                        msk = k < cnt
                        j = jnp.minimum(start + k, NPAD - 1)
                        col = plsc.load_gather(scol_v, [j], mask=msk)
                        val = plsc.load_gather(sval_v, [j], mask=msk)
                        plsc.addupdate_scatter(rows[b], [lanes, col], val, mask=msk)
                        return cc

                    lax.fori_loop(0, m, delta_body, 0)
                pltpu.async_copy(
                    rows[b], out_hbm.at[pl.ds(out_base + t * CHUNK, CHUNK)], so[b]
                )

        def turn(i, c):
            for b in range(2):
                t = 2 * i + b
                start_gather(t, b)
                finish_chunk(t - 1, 1 - b)
            return c

        lax.fori_loop(0, (n_chunks + 2) // 2, turn, 0)
        pltpu.make_async_copy(rows0, out_hbm.at[pl.ds(out_base, CHUNK)], so0).wait()
        pltpu.make_async_copy(rows1, out_hbm.at[pl.ds(out_base, CHUNK)], so1).wait()

    return gather_add


def kernel(x, W, spectrum, row_idx, col_idx):
    bsz, hist = x.shape
    n_tok = bsz * hist
    xf = jnp.transpose(x).reshape(n_tok).astype(jnp.int32)
    n_freq = row_idx.shape[0]

    iota = jnp.arange(n_freq, dtype=jnp.int32)
    srow, order = lax.sort_key_val(row_idx.astype(jnp.int32), iota)
    is_start = jnp.concatenate([jnp.ones((1,), bool), srow[1:] != srow[:-1]])
    first = lax.cummax(jnp.where(is_start, iota, 0))
    is_end = jnp.concatenate([srow[:-1] != srow[1:], jnp.ones((1,), bool)])
    last = jnp.flip(lax.cummin(jnp.flip(jnp.where(is_end, iota, n_freq - 1))))
    packed = first * 2048 + (last - first + 1)

    pad = NPAD - n_freq
    srow_p = jnp.pad(srow, (0, pad), constant_values=VOCAB_PAD)
    packed_p = jnp.pad(packed, (0, pad))
    order_p = jnp.pad(order, (0, pad))
    col_p = jnp.pad(col_idx.astype(jnp.int32), (0, pad))
    val_p = jnp.pad(spectrum.astype(jnp.float32) * SCALING, (0, pad))

    rid, scol_p, sval_p = _build_rid(srow_p, packed_p, order_p, col_p, val_p)
    xf2 = xf.reshape(n_tok // CHUNK, CHUNK)
    out = _make_gather_add(n_tok)(xf2, rid, W, scol_p, sval_p)
    return jnp.transpose(out.reshape(hist, bsz, DIM), (1, 0, 2))


# speedup vs baseline: 16.4319x; 1.0003x over previous
import functools

import jax
import jax.numpy as jnp
from jax import lax
from jax.experimental import pallas as pl
from jax.experimental.pallas import tpu as pltpu
from jax.experimental.pallas import tpu_sc as plsc

VOCAB = 100000
VOCAB_PAD = 102400
DIM = 128
SCALING = 1.0
NPAD = 1024
NC, NS, L = 2, 16, 16
NW = NC * NS
CHUNK = 128


def _wid():
    return lax.axis_index("s") * NC + lax.axis_index("c")


def _mesh():
    return plsc.VectorSubcoreMesh(core_axis_name="c", subcore_axis_name="s")


_CP = pltpu.CompilerParams(needs_layout_passes=False, use_tc_tiling_on_sc=False)


_RID_SLICE = VOCAB_PAD // NW


@functools.partial(
    pl.kernel,
    out_type=(
        jax.ShapeDtypeStruct((VOCAB_PAD,), jnp.int32),
        jax.ShapeDtypeStruct((NPAD,), jnp.int32),
        jax.ShapeDtypeStruct((NPAD,), jnp.float32),
    ),
    mesh=_mesh(),
    compiler_params=_CP,
    scratch_types=[
        pltpu.VMEM((_RID_SLICE,), jnp.int32),
        pltpu.VMEM((NPAD,), jnp.int32),
        pltpu.VMEM((NPAD,), jnp.int32),
        pltpu.VMEM((CHUNK,), jnp.int32),
        pltpu.VMEM((CHUNK,), jnp.int32),
        pltpu.VMEM((CHUNK,), jnp.float32),
        pltpu.SemaphoreType.DMA,
        pltpu.SemaphoreType.DMA,
    ],
)
def _build_rid(srow_hbm, pval_hbm, order_hbm, col_hbm, val_hbm,
               rid_hbm, scol_hbm, sval_hbm,
               rid_v, srow_v, pval_v, ord_v, colg_v, valg_v, sem0, sem1):
    wid = _wid()
    lo = wid * _RID_SLICE

    @pl.when(wid < NPAD // CHUNK)
    def _():
        pltpu.sync_copy(order_hbm.at[pl.ds(wid * CHUNK, CHUNK)], ord_v)
        pltpu.async_copy(col_hbm.at[ord_v], colg_v, sem0)
        pltpu.async_copy(val_hbm.at[ord_v], valg_v, sem1)

    def zero_body(i, c):
        rid_v[pl.ds(i * L, L)] = jnp.zeros((L,), jnp.int32)
        return c

    lax.fori_loop(0, _RID_SLICE // L, zero_body, 0)
    pltpu.sync_copy(srow_hbm, srow_v)
    pltpu.sync_copy(pval_hbm, pval_v)

    @pl.when(wid < NPAD // CHUNK)
    def _():
        pltpu.make_async_copy(col_hbm.at[ord_v], colg_v, sem0).wait()
        pltpu.make_async_copy(val_hbm.at[ord_v], valg_v, sem1).wait()
        pltpu.sync_copy(colg_v, scol_hbm.at[pl.ds(wid * CHUNK, CHUNK)])
        pltpu.sync_copy(valg_v, sval_hbm.at[pl.ds(wid * CHUNK, CHUNK)])

    def scat_body(i, c):
        off = i * L
        rows = srow_v[pl.ds(off, L)] - lo
        vals = pval_v[pl.ds(off, L)]
        msk = (rows >= 0) & (rows < _RID_SLICE)
        plsc.store_scatter(rid_v, [rows], vals, mask=msk)
        return c

    lax.fori_loop(0, NPAD // L, scat_body, 0)
    pltpu.sync_copy(rid_v, rid_hbm.at[pl.ds(lo, _RID_SLICE)])


NBUF = 4


def _make_gather_add(n_tok):
    b_per_w = n_tok // NW
    n_chunks = b_per_w // CHUNK

    @functools.partial(
        pl.kernel,
        out_type=jax.ShapeDtypeStruct((n_tok, DIM), jnp.float32),
        mesh=_mesh(),
        compiler_params=_CP,
        scratch_types=(
            [pltpu.VMEM((n_chunks, CHUNK), jnp.int32)]
            + [pltpu.VMEM((CHUNK,), jnp.int32)] * NBUF
            + [pltpu.VMEM((CHUNK, DIM), jnp.float32)] * NBUF
            + [
                pltpu.VMEM((NPAD,), jnp.int32),
                pltpu.VMEM((NPAD,), jnp.float32),
            ]
            + [pltpu.SemaphoreType.DMA] * (3 * NBUF)
        ),
    )
    def gather_add(xf2_hbm, rid_hbm, w_hbm, scol_hbm, sval_hbm, out_hbm,
                   idx2_v, *bufs):
        ridv = bufs[0:NBUF]
        rows = bufs[NBUF:2 * NBUF]
        scol_v = bufs[2 * NBUF]
        sval_v = bufs[2 * NBUF + 1]
        sw = bufs[2 * NBUF + 2:3 * NBUF + 2]
        sr = bufs[3 * NBUF + 2:4 * NBUF + 2]
        so = bufs[4 * NBUF + 2:5 * NBUF + 2]
        wid = _wid()
        pltpu.sync_copy(scol_hbm, scol_v)
        pltpu.sync_copy(sval_hbm, sval_v)
        pltpu.sync_copy(xf2_hbm.at[pl.ds(wid * n_chunks, n_chunks)], idx2_v)
        out_base = wid * b_per_w

        def start_gather(t, b):
            @pl.when(t < n_chunks)
            def _():
                @pl.when(t >= NBUF)
                def _():
                    pltpu.make_async_copy(
                        rows[b], out_hbm.at[pl.ds(out_base, CHUNK)], so[b]
                    ).wait()
                pltpu.async_copy(w_hbm.at[idx2_v.at[t]], rows[b], sw[b])
                pltpu.async_copy(rid_hbm.at[idx2_v.at[t]], ridv[b], sr[b])

        def finish_chunk(t, b):
            @pl.when(jnp.logical_and(t >= 0, t < n_chunks))
            def _():
                pltpu.make_async_copy(
                    w_hbm.at[idx2_v.at[0]], rows[b], sw[b]
                ).wait()
                pltpu.make_async_copy(
                    rid_hbm.at[idx2_v.at[0]], ridv[b], sr[b]
                ).wait()
                for v in range(CHUNK // L):
                    rv = ridv[b][pl.ds(v * L, L)]
                    cnt = lax.bitwise_and(rv, 2047)
                    start = lax.shift_right_logical(rv, 11)
                    m = jnp.max(cnt)
                    lanes = lax.iota(jnp.int32, L) + (v * L)

                    def delta_body(k, cc, start=start, cnt=cnt, lanes=lanes, b=b):
                        msk = k < cnt
                        j = jnp.minimum(start + k, NPAD - 1)
                        col = plsc.load_gather(scol_v, [j], mask=msk)
                        val = plsc.load_gather(sval_v, [j], mask=msk)
                        plsc.addupdate_scatter(rows[b], [lanes, col], val, mask=msk)
                        return cc

                    lax.fori_loop(0, m, delta_body, 0)
                pltpu.async_copy(
                    rows[b], out_hbm.at[pl.ds(out_base + t * CHUNK, CHUNK)], so[b]
                )

        def turn(i, c):
            for b in range(NBUF):
                t = NBUF * i + b
                start_gather(t, b)
                finish_chunk(t - (NBUF - 1), (b + 1) % NBUF)
            return c

        lax.fori_loop(0, (n_chunks + 2 * (NBUF - 1)) // NBUF, turn, 0)
        for b in range(NBUF):
            pltpu.make_async_copy(
                rows[b], out_hbm.at[pl.ds(out_base, CHUNK)], so[b]
            ).wait()

    return gather_add


def kernel(x, W, spectrum, row_idx, col_idx):
    bsz, hist = x.shape
    n_tok = bsz * hist
    xf = jnp.transpose(x).reshape(n_tok).astype(jnp.int32)
    n_freq = row_idx.shape[0]

    iota = jnp.arange(n_freq, dtype=jnp.int32)
    srow, order = lax.sort_key_val(row_idx.astype(jnp.int32), iota)
    is_start = jnp.concatenate([jnp.ones((1,), bool), srow[1:] != srow[:-1]])
    first = lax.cummax(jnp.where(is_start, iota, 0))
    is_end = jnp.concatenate([srow[:-1] != srow[1:], jnp.ones((1,), bool)])
    last = jnp.flip(lax.cummin(jnp.flip(jnp.where(is_end, iota, n_freq - 1))))
    packed = first * 2048 + (last - first + 1)

    pad = NPAD - n_freq
    srow_p = jnp.pad(srow, (0, pad), constant_values=VOCAB_PAD)
    packed_p = jnp.pad(packed, (0, pad))
    order_p = jnp.pad(order, (0, pad))
    col_p = jnp.pad(col_idx.astype(jnp.int32), (0, pad))
    val_p = jnp.pad(spectrum.astype(jnp.float32) * SCALING, (0, pad))

    rid, scol_p, sval_p = _build_rid(srow_p, packed_p, order_p, col_p, val_p)
    xf2 = xf.reshape(n_tok // CHUNK, CHUNK)
    out = _make_gather_add(n_tok)(xf2, rid, W, scol_p, sval_p)
    return jnp.transpose(out.reshape(hist, bsz, DIM), (1, 0, 2))


# speedup vs baseline: 16.9190x; 1.0296x over previous
import functools

import jax
import jax.numpy as jnp
from jax import lax
from jax.experimental import pallas as pl
from jax.experimental.pallas import tpu as pltpu
from jax.experimental.pallas import tpu_sc as plsc

VOCAB = 100000
VOCAB_PAD = 102400
DIM = 128
SCALING = 1.0
NPAD = 1024
NC, NS, L = 2, 16, 16
NW = NC * NS
CHUNK = 128


def _wid():
    return lax.axis_index("s") * NC + lax.axis_index("c")


def _mesh():
    return plsc.VectorSubcoreMesh(core_axis_name="c", subcore_axis_name="s")


_CP = pltpu.CompilerParams(needs_layout_passes=False, use_tc_tiling_on_sc=False)


_RID_SLICE = VOCAB_PAD // NW


@functools.partial(
    pl.kernel,
    out_type=(
        jax.ShapeDtypeStruct((VOCAB_PAD,), jnp.int32),
        jax.ShapeDtypeStruct((NPAD,), jnp.int32),
        jax.ShapeDtypeStruct((NPAD,), jnp.float32),
    ),
    mesh=_mesh(),
    compiler_params=_CP,
    scratch_types=[
        pltpu.VMEM((_RID_SLICE,), jnp.int32),
        pltpu.VMEM((NPAD,), jnp.int32),
        pltpu.VMEM((NPAD,), jnp.int32),
        pltpu.VMEM((CHUNK,), jnp.int32),
        pltpu.VMEM((CHUNK,), jnp.int32),
        pltpu.VMEM((CHUNK,), jnp.float32),
        pltpu.SemaphoreType.DMA,
        pltpu.SemaphoreType.DMA,
    ],
)
def _build_rid(srow_hbm, pval_hbm, order_hbm, col_hbm, val_hbm,
               rid_hbm, scol_hbm, sval_hbm,
               rid_v, srow_v, pval_v, ord_v, colg_v, valg_v, sem0, sem1):
    wid = _wid()
    lo = wid * _RID_SLICE

    @pl.when(wid < NPAD // CHUNK)
    def _():
        pltpu.sync_copy(order_hbm.at[pl.ds(wid * CHUNK, CHUNK)], ord_v)
        pltpu.async_copy(col_hbm.at[ord_v], colg_v, sem0)
        pltpu.async_copy(val_hbm.at[ord_v], valg_v, sem1)

    def zero_body(i, c):
        rid_v[pl.ds(i * L, L)] = jnp.zeros((L,), jnp.int32)
        return c

    lax.fori_loop(0, _RID_SLICE // L, zero_body, 0)
    pltpu.sync_copy(srow_hbm, srow_v)
    pltpu.sync_copy(pval_hbm, pval_v)

    @pl.when(wid < NPAD // CHUNK)
    def _():
        pltpu.make_async_copy(col_hbm.at[ord_v], colg_v, sem0).wait()
        pltpu.make_async_copy(val_hbm.at[ord_v], valg_v, sem1).wait()
        pltpu.sync_copy(colg_v, scol_hbm.at[pl.ds(wid * CHUNK, CHUNK)])
        pltpu.sync_copy(valg_v, sval_hbm.at[pl.ds(wid * CHUNK, CHUNK)])

    def scat_body(i, c):
        off = i * L
        rows = srow_v[pl.ds(off, L)] - lo
        vals = pval_v[pl.ds(off, L)]
        msk = (rows >= 0) & (rows < _RID_SLICE)
        plsc.store_scatter(rid_v, [rows], vals, mask=msk)
        return c

    lax.fori_loop(0, NPAD // L, scat_body, 0)
    pltpu.sync_copy(rid_v, rid_hbm.at[pl.ds(lo, _RID_SLICE)])


NBUF = 4


def _make_gather_add(n_tok):
    b_per_w = n_tok // NW
    n_chunks = b_per_w // CHUNK

    @functools.partial(
        pl.kernel,
        out_type=jax.ShapeDtypeStruct((n_tok, DIM), jnp.float32),
        mesh=_mesh(),
        compiler_params=_CP,
        scratch_types=(
            [pltpu.VMEM((n_chunks, CHUNK), jnp.int32)]
            + [pltpu.VMEM((CHUNK,), jnp.int32)] * NBUF
            + [pltpu.VMEM((CHUNK, DIM), jnp.float32)] * NBUF
            + [
                pltpu.VMEM((NPAD,), jnp.int32),
                pltpu.VMEM((NPAD,), jnp.float32),
            ]
            + [pltpu.SemaphoreType.DMA] * (3 * NBUF)
            + [pltpu.VMEM_SHARED((VOCAB_PAD,), jnp.int32)]
        ),
    )
    def gather_add(xf2_hbm, rid_hbm, w_hbm, scol_hbm, sval_hbm, out_hbm,
                   idx2_v, *bufs):
        ridv = bufs[0:NBUF]
        rows = bufs[NBUF:2 * NBUF]
        scol_v = bufs[2 * NBUF]
        sval_v = bufs[2 * NBUF + 1]
        sw = bufs[2 * NBUF + 2:3 * NBUF + 2]
        sr = bufs[3 * NBUF + 2:4 * NBUF + 2]
        so = bufs[4 * NBUF + 2:5 * NBUF + 2]
        rid_sh = bufs[5 * NBUF + 2]
        wid = _wid()
        @pl.when(lax.axis_index("s") == 0)
        def _():
            pltpu.sync_copy(rid_hbm, rid_sh)

        pltpu.sync_copy(scol_hbm, scol_v)
        pltpu.sync_copy(sval_hbm, sval_v)
        pltpu.sync_copy(xf2_hbm.at[pl.ds(wid * n_chunks, n_chunks)], idx2_v)
        plsc.subcore_barrier()
        out_base = wid * b_per_w

        def start_gather(t, b):
            @pl.when(t < n_chunks)
            def _():
                @pl.when(t >= NBUF)
                def _():
                    pltpu.make_async_copy(
                        rows[b], out_hbm.at[pl.ds(out_base, CHUNK)], so[b]
                    ).wait()
                pltpu.async_copy(w_hbm.at[idx2_v.at[t]], rows[b], sw[b])
                pltpu.async_copy(rid_sh.at[idx2_v.at[t]], ridv[b], sr[b])

        def finish_chunk(t, b):
            @pl.when(jnp.logical_and(t >= 0, t < n_chunks))
            def _():
                pltpu.make_async_copy(
                    w_hbm.at[idx2_v.at[0]], rows[b], sw[b]
                ).wait()
                pltpu.make_async_copy(
                    rid_sh.at[idx2_v.at[0]], ridv[b], sr[b]
                ).wait()
                for v in range(CHUNK // L):
                    rv = ridv[b][pl.ds(v * L, L)]
                    cnt = lax.bitwise_and(rv, 2047)
                    start = lax.shift_right_logical(rv, 11)
                    m = jnp.max(cnt)
                    lanes = lax.iota(jnp.int32, L) + (v * L)

                    def delta_body(k, cc, start=start, cnt=cnt, lanes=lanes, b=b):
                        msk = k < cnt
                        j = jnp.minimum(start + k, NPAD - 1)
                        col = plsc.load_gather(scol_v, [j], mask=msk)
                        val = plsc.load_gather(sval_v, [j], mask=msk)
                        plsc.addupdate_scatter(rows[b], [lanes, col], val, mask=msk)
                        return cc

                    lax.fori_loop(0, m, delta_body, 0)
                pltpu.async_copy(
                    rows[b], out_hbm.at[pl.ds(out_base + t * CHUNK, CHUNK)], so[b]
                )

        def turn(i, c):
            for b in range(NBUF):
                t = NBUF * i + b
                start_gather(t, b)
                finish_chunk(t - (NBUF - 1), (b + 1) % NBUF)
            return c

        lax.fori_loop(0, (n_chunks + 2 * (NBUF - 1)) // NBUF, turn, 0)
        for b in range(NBUF):
            pltpu.make_async_copy(
                rows[b], out_hbm.at[pl.ds(out_base, CHUNK)], so[b]
            ).wait()

    return gather_add


def kernel(x, W, spectrum, row_idx, col_idx):
    bsz, hist = x.shape
    n_tok = bsz * hist
    xf = jnp.transpose(x).reshape(n_tok).astype(jnp.int32)
    n_freq = row_idx.shape[0]

    iota = jnp.arange(n_freq, dtype=jnp.int32)
    srow, order = lax.sort_key_val(row_idx.astype(jnp.int32), iota)
    is_start = jnp.concatenate([jnp.ones((1,), bool), srow[1:] != srow[:-1]])
    first = lax.cummax(jnp.where(is_start, iota, 0))
    is_end = jnp.concatenate([srow[:-1] != srow[1:], jnp.ones((1,), bool)])
    last = jnp.flip(lax.cummin(jnp.flip(jnp.where(is_end, iota, n_freq - 1))))
    packed = first * 2048 + (last - first + 1)

    pad = NPAD - n_freq
    srow_p = jnp.pad(srow, (0, pad), constant_values=VOCAB_PAD)
    packed_p = jnp.pad(packed, (0, pad))
    order_p = jnp.pad(order, (0, pad))
    col_p = jnp.pad(col_idx.astype(jnp.int32), (0, pad))
    val_p = jnp.pad(spectrum.astype(jnp.float32) * SCALING, (0, pad))

    rid, scol_p, sval_p = _build_rid(srow_p, packed_p, order_p, col_p, val_p)
    xf2 = xf.reshape(n_tok // CHUNK, CHUNK)
    out = _make_gather_add(n_tok)(xf2, rid, W, scol_p, sval_p)
    return jnp.transpose(out.reshape(hist, bsz, DIM), (1, 0, 2))


# speedup vs baseline: 17.3105x; 1.0231x over previous
import functools

import jax
import jax.numpy as jnp
from jax import lax
from jax.experimental import pallas as pl
from jax.experimental.pallas import tpu as pltpu
from jax.experimental.pallas import tpu_sc as plsc

VOCAB = 100000
VOCAB_PAD = 102400
DIM = 128
SCALING = 1.0
NPAD = 1024
NC, NS, L = 2, 16, 16
NW = NC * NS
CHUNK = 128

_RID_SLICE = VOCAB_PAD // NS


def _wid():
    return lax.axis_index("s") * NC + lax.axis_index("c")


def _mesh():
    return plsc.VectorSubcoreMesh(core_axis_name="c", subcore_axis_name="s")


_CP = pltpu.CompilerParams(needs_layout_passes=False, use_tc_tiling_on_sc=False)


def _make_fused(n_tok):
    b_per_w = n_tok // NW
    n_chunks = b_per_w // CHUNK

    @functools.partial(
        pl.kernel,
        out_type=jax.ShapeDtypeStruct((n_tok, DIM), jnp.float32),
        mesh=_mesh(),
        compiler_params=_CP,
        scratch_types=[
            pltpu.VMEM((n_chunks, CHUNK), jnp.int32),
            pltpu.VMEM((CHUNK,), jnp.int32),
            pltpu.VMEM((CHUNK,), jnp.int32),
            pltpu.VMEM((CHUNK, DIM), jnp.float32),
            pltpu.VMEM((CHUNK, DIM), jnp.float32),
            pltpu.VMEM((NPAD,), jnp.int32),
            pltpu.VMEM((NPAD,), jnp.float32),
            pltpu.VMEM((_RID_SLICE,), jnp.int32),
            pltpu.VMEM((NPAD,), jnp.int32),
            pltpu.VMEM((NPAD,), jnp.int32),
            pltpu.VMEM((CHUNK,), jnp.int32),
            pltpu.VMEM((CHUNK,), jnp.int32),
            pltpu.VMEM((CHUNK,), jnp.float32),
            pltpu.SemaphoreType.DMA,
            pltpu.SemaphoreType.DMA,
            pltpu.SemaphoreType.DMA,
            pltpu.SemaphoreType.DMA,
            pltpu.SemaphoreType.DMA,
            pltpu.SemaphoreType.DMA,
            pltpu.SemaphoreType.DMA,
            pltpu.SemaphoreType.DMA,
            pltpu.VMEM_SHARED((VOCAB_PAD,), jnp.int32),
            pltpu.VMEM_SHARED((NPAD,), jnp.int32),
            pltpu.VMEM_SHARED((NPAD,), jnp.float32),
        ],
    )
    def fused(xf2_hbm, w_hbm, srow_hbm, pval_hbm, order_hbm, col_hbm, val_hbm,
              out_hbm,
              idx2_v, ridv0, ridv1, rows0, rows1, scol_v, sval_v,
              rid_loc, srow_v, pval_v, ord_v, colg_v, valg_v,
              sw0, sw1, sr0, sr1, so0, so1, sg0, sg1,
              rid_sh, scol_sh, sval_sh):
        wid = _wid()
        s = lax.axis_index("s")
        rows = (rows0, rows1)
        ridv = (ridv0, ridv1)
        sw = (sw0, sw1)
        sr = (sr0, sr1)
        so = (so0, so1)
        out_base = wid * b_per_w

        pltpu.sync_copy(xf2_hbm.at[pl.ds(wid * n_chunks, n_chunks)], idx2_v)
        for b in range(2):
            pltpu.async_copy(w_hbm.at[idx2_v.at[b]], rows[b], sw[b])

        lo = s * _RID_SLICE

        def zero_body(i, c):
            rid_loc[pl.ds(i * L, L)] = jnp.zeros((L,), jnp.int32)
            return c

        lax.fori_loop(0, _RID_SLICE // L, zero_body, 0)
        pltpu.sync_copy(srow_hbm, srow_v)
        pltpu.sync_copy(pval_hbm, pval_v)

        def scat_body(i, c):
            off = i * L
            r = srow_v[pl.ds(off, L)] - lo
            v = pval_v[pl.ds(off, L)]
            msk = (r >= 0) & (r < _RID_SLICE)
            plsc.store_scatter(rid_loc, [r], v, mask=msk)
            return c

        lax.fori_loop(0, NPAD // L, scat_body, 0)
        pltpu.sync_copy(rid_loc, rid_sh.at[pl.ds(lo, _RID_SLICE)])

        @pl.when(s < NPAD // CHUNK)
        def _():
            pltpu.sync_copy(order_hbm.at[pl.ds(s * CHUNK, CHUNK)], ord_v)
            pltpu.async_copy(col_hbm.at[ord_v], colg_v, sg0)
            pltpu.async_copy(val_hbm.at[ord_v], valg_v, sg1)
            pltpu.make_async_copy(col_hbm.at[ord_v], colg_v, sg0).wait()
            pltpu.make_async_copy(val_hbm.at[ord_v], valg_v, sg1).wait()
            pltpu.sync_copy(colg_v, scol_sh.at[pl.ds(s * CHUNK, CHUNK)])
            pltpu.sync_copy(valg_v, sval_sh.at[pl.ds(s * CHUNK, CHUNK)])

        plsc.subcore_barrier()
        pltpu.sync_copy(scol_sh, scol_v)
        pltpu.sync_copy(sval_sh, sval_v)

        def start_gather(t, b):
            @pl.when(t < n_chunks)
            def _():
                @pl.when(t >= 2)
                def _():
                    pltpu.make_async_copy(
                        rows[b], out_hbm.at[pl.ds(out_base, CHUNK)], so[b]
                    ).wait()
                pltpu.async_copy(w_hbm.at[idx2_v.at[t]], rows[b], sw[b])
                pltpu.async_copy(rid_sh.at[idx2_v.at[t]], ridv[b], sr[b])

        def finish_chunk(t, b):
            @pl.when(jnp.logical_and(t >= 0, t < n_chunks))
            def _():
                pltpu.make_async_copy(
                    w_hbm.at[idx2_v.at[0]], rows[b], sw[b]
                ).wait()
                pltpu.make_async_copy(
                    rid_sh.at[idx2_v.at[0]], ridv[b], sr[b]
                ).wait()
                for v in range(CHUNK // L):
                    rv = ridv[b][pl.ds(v * L, L)]
                    cnt = lax.bitwise_and(rv, 2047)
                    start = lax.shift_right_logical(rv, 11)
                    m = jnp.max(cnt)
                    lanes = lax.iota(jnp.int32, L) + (v * L)

                    def delta_body(k, cc, start=start, cnt=cnt, lanes=lanes, b=b):
                        msk = k < cnt
                        j = jnp.minimum(start + k, NPAD - 1)
                        col = plsc.load_gather(scol_v, [j], mask=msk)
                        val = plsc.load_gather(sval_v, [j], mask=msk)
                        plsc.addupdate_scatter(rows[b], [lanes, col], val, mask=msk)
                        return cc

                    lax.fori_loop(0, m, delta_body, 0)
                pltpu.async_copy(
                    rows[b], out_hbm.at[pl.ds(out_base + t * CHUNK, CHUNK)], so[b]
                )

        pltpu.async_copy(rid_sh.at[idx2_v.at[0]], ridv0, sr0)
        pltpu.async_copy(rid_sh.at[idx2_v.at[1]], ridv1, sr1)
        finish_chunk(0, 0)

        def turn(i, c):
            for b in range(2):
                t = 2 * i + b
                start_gather(t, b)
                finish_chunk(t - 1, 1 - b)
            return c

        lax.fori_loop(1, (n_chunks + 2) // 2, turn, 0)
        pltpu.make_async_copy(rows0, out_hbm.at[pl.ds(out_base, CHUNK)], so0).wait()
        pltpu.make_async_copy(rows1, out_hbm.at[pl.ds(out_base, CHUNK)], so1).wait()

    return fused


def kernel(x, W, spectrum, row_idx, col_idx):
    bsz, hist = x.shape
    n_tok = bsz * hist
    xf = jnp.transpose(x).reshape(n_tok).astype(jnp.int32)
    n_freq = row_idx.shape[0]

    iota = jnp.arange(n_freq, dtype=jnp.int32)
    srow, order = lax.sort_key_val(row_idx.astype(jnp.int32), iota)
    is_start = jnp.concatenate([jnp.ones((1,), bool), srow[1:] != srow[:-1]])
    first = lax.cummax(jnp.where(is_start, iota, 0))
    is_end = jnp.concatenate([srow[:-1] != srow[1:], jnp.ones((1,), bool)])
    last = jnp.flip(lax.cummin(jnp.flip(jnp.where(is_end, iota, n_freq - 1))))
    packed = first * 2048 + (last - first + 1)

    pad = NPAD - n_freq
    srow_p = jnp.pad(srow, (0, pad), constant_values=VOCAB_PAD)
    packed_p = jnp.pad(packed, (0, pad))
    order_p = jnp.pad(order, (0, pad))
    col_p = jnp.pad(col_idx.astype(jnp.int32), (0, pad))
    val_p = jnp.pad(spectrum.astype(jnp.float32) * SCALING, (0, pad))

    xf2 = xf.reshape(n_tok // CHUNK, CHUNK)
    out = _make_fused(n_tok)(xf2, W, srow_p, packed_p, order_p, col_p, val_p)
    return jnp.transpose(out.reshape(hist, bsz, DIM), (1, 0, 2))


# speedup vs baseline: 17.4338x; 1.0071x over previous
import functools

import jax
import jax.numpy as jnp
from jax import lax
from jax.experimental import pallas as pl
from jax.experimental.pallas import tpu as pltpu
from jax.experimental.pallas import tpu_sc as plsc

VOCAB = 100000
VOCAB_PAD = 102400
DIM = 128
SCALING = 1.0
NPAD = 1024
NC, NS, L = 2, 16, 16
NW = NC * NS
CHUNK = 128

_RID_SLICE = VOCAB_PAD // NS


def _wid():
    return lax.axis_index("s") * NC + lax.axis_index("c")


def _mesh():
    return plsc.VectorSubcoreMesh(core_axis_name="c", subcore_axis_name="s")


_CP = pltpu.CompilerParams(needs_layout_passes=False, use_tc_tiling_on_sc=False)


def _make_fused(n_tok):
    b_per_w = n_tok // NW
    n_chunks = b_per_w // CHUNK

    @functools.partial(
        pl.kernel,
        out_type=jax.ShapeDtypeStruct((n_tok, DIM), jnp.float32),
        mesh=_mesh(),
        compiler_params=_CP,
        scratch_types=[
            pltpu.VMEM((n_chunks, CHUNK), jnp.int32),
            pltpu.VMEM((CHUNK,), jnp.int32),
            pltpu.VMEM((CHUNK,), jnp.int32),
            pltpu.VMEM((CHUNK, DIM), jnp.float32),
            pltpu.VMEM((CHUNK, DIM), jnp.float32),
            pltpu.VMEM((NPAD,), jnp.int32),
            pltpu.VMEM((NPAD,), jnp.float32),
            pltpu.VMEM((_RID_SLICE,), jnp.int32),
            pltpu.VMEM((NPAD,), jnp.int32),
            pltpu.VMEM((NPAD,), jnp.int32),
            pltpu.VMEM((CHUNK,), jnp.int32),
            pltpu.VMEM((CHUNK,), jnp.int32),
            pltpu.VMEM((CHUNK,), jnp.float32),
            pltpu.SemaphoreType.DMA,
            pltpu.SemaphoreType.DMA,
            pltpu.SemaphoreType.DMA,
            pltpu.SemaphoreType.DMA,
            pltpu.SemaphoreType.DMA,
            pltpu.SemaphoreType.DMA,
            pltpu.SemaphoreType.DMA,
            pltpu.SemaphoreType.DMA,
            pltpu.VMEM_SHARED((VOCAB_PAD,), jnp.int32),
            pltpu.VMEM_SHARED((NPAD,), jnp.int32),
            pltpu.VMEM_SHARED((NPAD,), jnp.float32),
        ],
    )
    def fused(xf2_hbm, w_hbm, srow_hbm, pval_hbm, order_hbm, col_hbm, val_hbm,
              out_hbm,
              idx2_v, ridv0, ridv1, rows0, rows1, scol_v, sval_v,
              rid_loc, srow_v, pval_v, ord_v, colg_v, valg_v,
              sw0, sw1, sr0, sr1, so0, so1, sg0, sg1,
              rid_sh, scol_sh, sval_sh):
        wid = _wid()
        s = lax.axis_index("s")
        rows = (rows0, rows1)
        ridv = (ridv0, ridv1)
        sw = (sw0, sw1)
        sr = (sr0, sr1)
        so = (so0, so1)
        out_base = wid * b_per_w

        pltpu.sync_copy(xf2_hbm.at[pl.ds(wid * n_chunks, n_chunks)], idx2_v)
        for b in range(2):
            pltpu.async_copy(w_hbm.at[idx2_v.at[b]], rows[b], sw[b])

        lo = s * _RID_SLICE

        def zero_body(i, c):
            rid_loc[pl.ds(i * L, L)] = jnp.zeros((L,), jnp.int32)
            return c

        lax.fori_loop(0, _RID_SLICE // L, zero_body, 0)
        pltpu.sync_copy(srow_hbm, srow_v)
        pltpu.sync_copy(pval_hbm, pval_v)

        def scat_body(i, c):
            off = i * L
            r = srow_v[pl.ds(off, L)] - lo
            v = pval_v[pl.ds(off, L)]
            msk = (r >= 0) & (r < _RID_SLICE)
            plsc.store_scatter(rid_loc, [r], v, mask=msk)
            return c

        lax.fori_loop(0, NPAD // L, scat_body, 0)
        pltpu.sync_copy(rid_loc, rid_sh.at[pl.ds(lo, _RID_SLICE)])

        @pl.when(s < NPAD // CHUNK)
        def _():
            pltpu.sync_copy(order_hbm.at[pl.ds(s * CHUNK, CHUNK)], ord_v)
            pltpu.async_copy(col_hbm.at[ord_v], colg_v, sg0)
            pltpu.async_copy(val_hbm.at[ord_v], valg_v, sg1)
            pltpu.make_async_copy(col_hbm.at[ord_v], colg_v, sg0).wait()
            pltpu.make_async_copy(val_hbm.at[ord_v], valg_v, sg1).wait()
            pltpu.sync_copy(colg_v, scol_sh.at[pl.ds(s * CHUNK, CHUNK)])
            pltpu.sync_copy(valg_v, sval_sh.at[pl.ds(s * CHUNK, CHUNK)])

        plsc.subcore_barrier()
        pltpu.sync_copy(scol_sh, scol_v)
        pltpu.sync_copy(sval_sh, sval_v)

        def start_gather(t, b):
            @pl.when(t < n_chunks)
            def _():
                @pl.when(t >= 2)
                def _():
                    pltpu.make_async_copy(
                        rows[b], out_hbm.at[pl.ds(out_base, CHUNK)], so[b]
                    ).wait()
                pltpu.async_copy(w_hbm.at[idx2_v.at[t]], rows[b], sw[b])
                pltpu.async_copy(rid_sh.at[idx2_v.at[t]], ridv[b], sr[b])

        def finish_chunk(t, b):
            @pl.when(jnp.logical_and(t >= 0, t < n_chunks))
            def _():
                pltpu.make_async_copy(
                    w_hbm.at[idx2_v.at[0]], rows[b], sw[b]
                ).wait()
                pltpu.make_async_copy(
                    rid_sh.at[idx2_v.at[0]], ridv[b], sr[b]
                ).wait()
                for v in range(CHUNK // L):
                    rv = ridv[b][pl.ds(v * L, L)]
                    cnt = lax.bitwise_and(rv, 2047)
                    start = lax.shift_right_logical(rv, 11)
                    m = jnp.max(cnt)
                    lanes = lax.iota(jnp.int32, L) + (v * L)

                    def delta_body(k, cc, start=start, cnt=cnt, lanes=lanes, b=b):
                        msk = k < cnt
                        j = jnp.minimum(start + k, NPAD - 1)
                        col = plsc.load_gather(scol_v, [j], mask=msk)
                        val = plsc.load_gather(sval_v, [j], mask=msk)
                        plsc.addupdate_scatter(rows[b], [lanes, col], val, mask=msk)
                        return cc

                    lax.fori_loop(0, m, delta_body, 0)
                pltpu.async_copy(
                    rows[b], out_hbm.at[pl.ds(out_base + t * CHUNK, CHUNK)], so[b]
                )

        pltpu.async_copy(rid_sh.at[idx2_v.at[0]], ridv0, sr0)
        pltpu.async_copy(rid_sh.at[idx2_v.at[1]], ridv1, sr1)
        finish_chunk(0, 0)

        def turn(i, c):
            for b in range(2):
                t = 2 * i + b
                start_gather(t, b)
                finish_chunk(t - 1, 1 - b)
            return c

        lax.fori_loop(1, (n_chunks + 2) // 2, turn, 0)
        pltpu.make_async_copy(rows0, out_hbm.at[pl.ds(out_base, CHUNK)], so0).wait()
        pltpu.make_async_copy(rows1, out_hbm.at[pl.ds(out_base, CHUNK)], so1).wait()

    return fused


def kernel(x, W, spectrum, row_idx, col_idx):
    bsz, hist = x.shape
    n_tok = bsz * hist
    xf = jnp.transpose(x).reshape(n_tok).astype(jnp.int32)
    n_freq = row_idx.shape[0]

    pad = NPAD - n_freq
    iota = jnp.arange(NPAD, dtype=jnp.int32)
    row_p = jnp.pad(row_idx.astype(jnp.int32), (0, pad), constant_values=VOCAB_PAD)
    srow_p, order_p = lax.sort_key_val(row_p, iota)
    is_start = jnp.concatenate([jnp.ones((1,), bool), srow_p[1:] != srow_p[:-1]])
    first = lax.cummax(jnp.where(is_start, iota, 0))
    is_end = jnp.concatenate([srow_p[:-1] != srow_p[1:], jnp.ones((1,), bool)])
    last = jnp.flip(lax.cummin(jnp.flip(jnp.where(is_end, iota, NPAD - 1))))
    packed_p = first * 2048 + (last - first + 1)
    col_p = jnp.pad(col_idx.astype(jnp.int32), (0, pad))
    val_p = jnp.pad(spectrum.astype(jnp.float32) * SCALING, (0, pad))

    xf2 = xf.reshape(n_tok // CHUNK, CHUNK)
    out = _make_fused(n_tok)(xf2, W, srow_p, packed_p, order_p, col_p, val_p)
    return jnp.transpose(out.reshape(hist, bsz, DIM), (1, 0, 2))
